# baseline (speedup 1.0000x reference)
import functools

import jax
import jax.numpy as jnp
from jax import lax
from jax.experimental import pallas as pl
from jax.experimental.pallas import tpu as pltpu
from jax.experimental.pallas import tpu_sc as plsc

_B = 16384
_D = 64
_NW = 32
_BPW = _B // _NW
_IC = 128
_NC = _BPW // _IC


def _build():
    mesh = plsc.VectorSubcoreMesh(core_axis_name="c", subcore_axis_name="s")

    @functools.partial(
        pl.kernel,
        mesh=mesh,
        compiler_params=pltpu.CompilerParams(
            needs_layout_passes=False, use_tc_tiling_on_sc=False),
        out_type=jax.ShapeDtypeStruct((_B,), jnp.float32),
        scratch_types=[
            pltpu.VMEM((_NC, _IC), jnp.int32),
            pltpu.VMEM((_NC, _IC), jnp.int32),
            pltpu.VMEM((_BPW, _D), jnp.float32),
            pltpu.VMEM((_BPW, _D), jnp.float32),
            pltpu.VMEM((_BPW,), jnp.float32),
            pltpu.VMEM((_BPW,), jnp.float32),
            pltpu.VMEM((_BPW,), jnp.float32),
            pltpu.SemaphoreType.DMA,
        ],
    )
    def body(user_hbm, item_hbm, uemb_hbm, iemb_hbm, ubias_hbm, ibias_hbm,
             out_hbm, uidx, iidx, urows, irows, ub, ib, outv, sem):
        wid = lax.axis_index("s") * 2 + lax.axis_index("c")
        base = wid * _BPW

        pltpu.sync_copy(user_hbm.at[pl.ds(wid * _NC, _NC)], uidx)
        pltpu.sync_copy(item_hbm.at[pl.ds(wid * _NC, _NC)], iidx)

        copies = []
        for j in range(_NC):
            sl = pl.ds(j * _IC, _IC)
            copies.append(pltpu.async_copy(uemb_hbm.at[uidx.at[j]], urows.at[sl], sem))
            copies.append(pltpu.async_copy(iemb_hbm.at[iidx.at[j]], irows.at[sl], sem))
            copies.append(pltpu.async_copy(ubias_hbm.at[uidx.at[j]], ub.at[sl], sem))
            copies.append(pltpu.async_copy(ibias_hbm.at[iidx.at[j]], ib.at[sl], sem))
        for c in copies:
            c.wait()

        lanes = lax.iota(jnp.int32, 16)

        def group(g, carry):
            rows = lanes + g * 16
            sl16 = pl.ds(g * 16, 16)
            acc = ub[sl16] + ib[sl16] + 3.5
            for d in range(_D):
                cols = lax.bitwise_and(lanes + d, _D - 1)
                acc = acc + (plsc.load_gather(urows, [rows, cols])
                             * plsc.load_gather(irows, [rows, cols]))
            outv[sl16] = acc
            return carry

        lax.fori_loop(0, _BPW // 16, group, 0)
        pltpu.sync_copy(outv, out_hbm.at[pl.ds(base, _BPW)])

    return body


_sc_call = _build()


def kernel(user, item, user_emb, item_emb, user_bias, item_bias):
    u2 = user.astype(jnp.int32).reshape(_NW * _NC, _IC)
    i2 = item.astype(jnp.int32).reshape(_NW * _NC, _IC)
    return _sc_call(u2, i2, user_emb, item_emb,
                    user_bias.sum(axis=1), item_bias.sum(axis=1))


# speedup vs baseline: 1.0006x; 1.0006x over previous
import functools

import jax
import jax.numpy as jnp
from jax import lax
from jax.experimental import pallas as pl
from jax.experimental.pallas import tpu as pltpu
from jax.experimental.pallas import tpu_sc as plsc

_B = 16384
_D = 64
_NW = 32
_BPW = _B // _NW
_IC = 128
_NC = _BPW // _IC


def _build():
    mesh = plsc.VectorSubcoreMesh(core_axis_name="c", subcore_axis_name="s")

    @functools.partial(
        pl.kernel,
        mesh=mesh,
        compiler_params=pltpu.CompilerParams(
            needs_layout_passes=False, use_tc_tiling_on_sc=False),
        out_type=jax.ShapeDtypeStruct((_B,), jnp.float32),
        scratch_types=[
            pltpu.VMEM((_BPW,), jnp.int32),
            pltpu.VMEM((_BPW,), jnp.int32),
            pltpu.VMEM((_BPW, _D), jnp.float32),
            pltpu.VMEM((_BPW, _D), jnp.float32),
            pltpu.VMEM((_BPW,), jnp.float32),
            pltpu.VMEM((_BPW,), jnp.float32),
            pltpu.VMEM((_BPW,), jnp.float32),
            pltpu.SemaphoreType.DMA,
        ],
    )
    def body(user_hbm, item_hbm, uemb_hbm, iemb_hbm, ubias_hbm, ibias_hbm,
             out_hbm, uidx, iidx, urows, irows, ub, ib, outv, sem):
        wid = lax.axis_index("s") * 2 + lax.axis_index("c")
        base = wid * _BPW

        pltpu.sync_copy(user_hbm.at[pl.ds(base, _BPW)], uidx)
        pltpu.sync_copy(item_hbm.at[pl.ds(base, _BPW)], iidx)

        copies = []
        for j in range(_NC):
            sl = pl.ds(j * _IC, _IC)
            copies.append(pltpu.async_copy(uemb_hbm.at[uidx.at[sl]], urows.at[sl], sem))
            copies.append(pltpu.async_copy(iemb_hbm.at[iidx.at[sl]], irows.at[sl], sem))
            copies.append(pltpu.async_copy(ubias_hbm.at[uidx.at[sl]], ub.at[sl], sem))
            copies.append(pltpu.async_copy(ibias_hbm.at[iidx.at[sl]], ib.at[sl], sem))
        for c in copies:
            c.wait()

        lanes = lax.iota(jnp.int32, 16)

        def group(g, carry):
            rows = lanes + g * 16
            sl16 = pl.ds(g * 16, 16)
            acc = ub[sl16] + ib[sl16] + 3.5
            for d in range(_D):
                cols = lax.bitwise_and(lanes + d, _D - 1)
                acc = acc + (plsc.load_gather(urows, [rows, cols])
                             * plsc.load_gather(irows, [rows, cols]))
            outv[sl16] = acc
            return carry

        lax.fori_loop(0, _BPW // 16, group, 0)
        pltpu.sync_copy(outv, out_hbm.at[pl.ds(base, _BPW)])

    return body


_sc_call = _build()


def kernel(user, item, user_emb, item_emb, user_bias, item_bias):
    return _sc_call(user.astype(jnp.int32), item.astype(jnp.int32),
                    user_emb, item_emb,
                    user_bias.sum(axis=1), item_bias.sum(axis=1))


# speedup vs baseline: 1.0338x; 1.0331x over previous
import functools

import jax
import jax.numpy as jnp
from jax import lax
from jax.experimental import pallas as pl
from jax.experimental.pallas import tpu as pltpu
from jax.experimental.pallas import tpu_sc as plsc

_B = 16384
_D = 64
_DP = 128
_NW = 32
_BPW = _B // _NW
_IC = 128
_NC = _BPW // _IC
_HALF = _BPW // 2


def _build():
    mesh = plsc.VectorSubcoreMesh(core_axis_name="c", subcore_axis_name="s")

    @functools.partial(
        pl.kernel,
        mesh=mesh,
        compiler_params=pltpu.CompilerParams(needs_layout_passes=False),
        out_type=jax.ShapeDtypeStruct((_B,), jnp.float32),
        scratch_types=[
            pltpu.VMEM((_BPW,), jnp.int32),
            pltpu.VMEM((_BPW,), jnp.int32),
            pltpu.VMEM((_HALF, _DP), jnp.float32),
            pltpu.VMEM((_HALF, _DP), jnp.float32),
            pltpu.VMEM((_BPW,), jnp.float32),
            pltpu.VMEM((_BPW,), jnp.float32),
            pltpu.VMEM((_BPW,), jnp.float32),
            pltpu.SemaphoreType.DMA,
            pltpu.SemaphoreType.DMA,
        ],
    )
    def body(user_hbm, item_hbm, uemb_hbm, iemb_hbm, ubias_hbm, ibias_hbm,
             out_hbm, uidx, iidx, urows, irows, ub, ib, outv, sem, bsem):
        wid = lax.axis_index("s") * 2 + lax.axis_index("c")
        base = wid * _BPW

        pltpu.sync_copy(user_hbm.at[pl.ds(base, _BPW)], uidx)
        pltpu.sync_copy(item_hbm.at[pl.ds(base, _BPW)], iidx)

        bias_copies = []
        for j in range(_NC):
            sl = pl.ds(j * _IC, _IC)
            bias_copies.append(
                pltpu.async_copy(ubias_hbm.at[uidx.at[sl]], ub.at[sl], bsem))
            bias_copies.append(
                pltpu.async_copy(ibias_hbm.at[iidx.at[sl]], ib.at[sl], bsem))

        lanes = lax.iota(jnp.int32, 16)

        def fire(h):
            cs = []
            for j in range(_HALF // _IC):
                isl = pl.ds(h * _HALF + j * _IC, _IC)
                dsl = pl.ds(j * _IC, _IC)
                cs.append(pltpu.async_copy(uemb_hbm.at[uidx.at[isl]],
                                           urows.at[dsl], sem))
                cs.append(pltpu.async_copy(iemb_hbm.at[iidx.at[isl]],
                                           irows.at[dsl], sem))
            return cs

        def compute(h):
            def group(g, carry):
                rows = lanes + g * 16
                sl16 = pl.ds(h * _HALF + g * 16, 16)
                acc = ub[sl16] + ib[sl16] + 3.5
                for d in range(_D):
                    cols = lax.bitwise_and(lanes + d, _D - 1)
                    acc = acc + (plsc.load_gather(urows, [rows, cols])
                                 * plsc.load_gather(irows, [rows, cols]))
                outv[sl16] = acc
                return carry

            lax.fori_loop(0, _HALF // 16, group, 0)

        for c in bias_copies:
            c.wait()

        for h in range(2):
            for c in fire(h):
                c.wait()
            compute(h)

        pltpu.sync_copy(outv, out_hbm.at[pl.ds(base, _BPW)])

    return body


_sc_call = _build()


def kernel(user, item, user_emb, item_emb, user_bias, item_bias):
    up = jnp.pad(user_emb, ((0, 0), (0, _DP - _D)))
    ip = jnp.pad(item_emb, ((0, 0), (0, _DP - _D)))
    return _sc_call(user.astype(jnp.int32), item.astype(jnp.int32),
                    up, ip, user_bias.sum(axis=1), item_bias.sum(axis=1))
